# baseline (speedup 1.0000x reference)
import functools

import jax
import jax.numpy as jnp
from jax import lax
from jax.experimental import pallas as pl
from jax.experimental.pallas import tpu as pltpu
from jax.experimental.pallas import tpu_sc as plsc

B, N, C = 32, 20000, 16
LANES = 16
ANCHOR_ITERS = N // LANES


def _tc_softplus_sum(*refs):
    o_ref = refs[-1]
    @pl.when(pl.program_id(0) == 0)
    def _init():
        o_ref[0, 0] = 0.0

    def sp_sum(x):
        u = jnp.exp(-jnp.abs(x))
        return jnp.sum(jnp.maximum(x, 0.0) + jnp.log(1.0 + u))

    acc = jnp.float32(0.0)
    for r in refs[:-1]:
        acc += sp_sum(r[...])
    o_ref[0, 0] += acc


def _sc_argmax_body(a_s_hbm, a_e_hbm, gt_hbm, cls_hbm, s_hbm, bb_hbm,
                    out_hbm, va_s, va_e, vgt, vcls, vrow, vbb0, vbb1, vout,
                    shared):
    sid = lax.axis_index("s")
    b = sid * 2 + lax.axis_index("c")

    @pl.when(sid == 0)
    def _stage():
        pltpu.sync_copy(a_s_hbm, shared.at[0])
        pltpu.sync_copy(a_e_hbm, shared.at[1])

    plsc.subcore_barrier()
    pltpu.sync_copy(shared.at[0], va_s)
    pltpu.sync_copy(shared.at[1], va_e)

    iota = lax.iota(jnp.int32, LANES)

    gt_off = (2 * b // LANES) * LANES
    pltpu.sync_copy(gt_hbm.at[pl.ds(gt_off, LANES)], vgt)
    gtv = vgt[...]
    r_gt = 2 * b - gt_off
    g0 = jnp.sum(jnp.where(iota == r_gt, gtv, 0.0))
    g1 = jnp.sum(jnp.where(iota == r_gt + 1, gtv, 0.0))
    garea = g1 - g0

    def iter_body(i, carry):
        run_max, run_idx = carry
        asv = va_s[pl.ds(i * LANES, LANES)]
        aev = va_e[pl.ds(i * LANES, LANES)]
        inter = jnp.maximum(jnp.minimum(aev, g1) - jnp.maximum(asv, g0), 0.0)
        union = (aev - asv) + garea - inter
        iou = inter / union
        better = iou > run_max
        return (jnp.where(better, iou, run_max),
                jnp.where(better, i * LANES + iota, run_idx))

    run_max, run_idx = lax.fori_loop(
        0, ANCHOR_ITERS, iter_body,
        (jnp.full((LANES,), -1.0, jnp.float32),
         jnp.zeros((LANES,), jnp.int32)))
    m = jnp.max(run_max)
    best = jnp.min(jnp.where(run_max == m, run_idx, jnp.int32(2 ** 30)))

    cls_off = (b // LANES) * LANES
    pltpu.sync_copy(cls_hbm.at[pl.ds(cls_off, LANES)], vcls)
    cls = jnp.max(jnp.where(iota == b - cls_off, vcls[...], 0))
    blk = (best // 128) * 128
    pltpu.sync_copy(s_hbm.at[b, cls, pl.ds(blk, 128)], vrow)
    pltpu.sync_copy(bb_hbm.at[0, b, pl.ds(blk, 128)], vbb0)
    pltpu.sync_copy(bb_hbm.at[1, b, pl.ds(blk, 128)], vbb1)
    lane = best - blk
    l16 = (lane // LANES) * LANES
    r = lane - l16
    sval = jnp.sum(jnp.where(iota == r, vrow[pl.ds(l16, LANES)], 0.0))
    d0 = jnp.sum(jnp.where(iota == r, vbb0[pl.ds(l16, LANES)], 0.0)) - g0
    d1 = jnp.sum(jnp.where(iota == r, vbb1[pl.ds(l16, LANES)], 0.0)) - g1

    def sl1(d):
        ad = jnp.abs(d)
        return jnp.where(ad < 1.0, 0.5 * d * d, ad - 0.5)

    corr = sl1(d0) + sl1(d1) - sval
    vout[...] = jnp.where(iota == 0, corr, 0.0)
    pltpu.sync_copy(vout, out_hbm.at[pl.ds(b * LANES, LANES)])


def kernel(scores, bboxes, gt_classes, gt_bboxes, anchors):
    scores_t = jnp.transpose(scores, (0, 2, 1))
    bb_t = jnp.transpose(bboxes, (2, 0, 1))

    tc_sum = pl.pallas_call(
        _tc_softplus_sum,
        grid=(B // 4,),
        in_specs=[pl.BlockSpec((1, 8, N),
                               functools.partial(
                                   lambda k, i: (4 * i + k // 2, k % 2, 0), k))
                  for k in range(8)],
        out_specs=pl.BlockSpec(memory_space=pltpu.SMEM),
        out_shape=jax.ShapeDtypeStruct((1, 1), jnp.float32),
    )(*([scores_t] * 8))

    mesh = plsc.VectorSubcoreMesh(core_axis_name="c", subcore_axis_name="s")
    sc_kernel = functools.partial(
        pl.kernel, mesh=mesh,
        compiler_params=pltpu.CompilerParams(needs_layout_passes=False),
        out_type=jax.ShapeDtypeStruct((B * LANES,), jnp.float32),
        scratch_types=[
            pltpu.VMEM((N,), jnp.float32),
            pltpu.VMEM((N,), jnp.float32),
            pltpu.VMEM((LANES,), jnp.float32),
            pltpu.VMEM((LANES,), jnp.int32),
            pltpu.VMEM((128,), jnp.float32),
            pltpu.VMEM((128,), jnp.float32),
            pltpu.VMEM((128,), jnp.float32),
            pltpu.VMEM((LANES,), jnp.float32),
            pltpu.VMEM_SHARED((2, N), jnp.float32),
        ],
    )(_sc_argmax_body)
    corr = sc_kernel(anchors[:, 0], anchors[:, 1], gt_bboxes.reshape(-1),
                     gt_classes, scores_t, bb_t)

    return tc_sum[0, 0] + jnp.sum(corr)
